# baseline (speedup 1.0000x reference)
import functools

import jax
import jax.numpy as jnp
from jax import lax
from jax.experimental import pallas as pl
from jax.experimental.pallas import tpu as pltpu
from jax.experimental.pallas import tpu_sc as plsc

N = 10000
E = 320000
C = 128
G = 16
EPS = 1e-5

NPAD = 10112
EBLK = 64
NTILES = 32
EP = 327680
NBLK = EP // EBLK
DEG_BPT = NBLK // NTILES
ZROWS = NPAD // 16
ICH = 8
NRING = 4
SPLIT0 = 3840
T0 = SPLIT0 // 16
T1 = (NBLK - SPLIT0) // 16


def _fill_f32(ref, rows, val):
    v = jnp.full((16,), val, jnp.float32)

    def body(i, _):
        for j in range(8):
            ref[i, pl.ds(j * 16, 16)] = v
        return 0

    lax.fori_loop(0, rows, body, 0)


@functools.cache
def _sc_kernels():
    mesh = plsc.VectorSubcoreMesh(core_axis_name="c", subcore_axis_name="s")
    deg = functools.partial(
        pl.kernel,
        out_type=jax.ShapeDtypeStruct((2 * NPAD,), jnp.float32),
        mesh=mesh,
        scratch_types=[
            pltpu.VMEM((DEG_BPT, EBLK), jnp.int32),
            pltpu.VMEM((128,), jnp.float32),
            pltpu.VMEM((EBLK,), jnp.float32),
            pltpu.VMEM((ZROWS,), jnp.float32),
            pltpu.VMEM_SHARED((NPAD,), jnp.float32),
        ],
    )(_deg_body)
    scat = functools.partial(
        pl.kernel,
        out_type=jax.ShapeDtypeStruct((2, NPAD, C), jnp.float32),
        mesh=mesh,
        scratch_types=[
            pltpu.VMEM((ICH, EBLK), jnp.int32),
            pltpu.VMEM((ICH, EBLK), jnp.int32),
            [pltpu.VMEM((EBLK, C), jnp.float32) for _ in range(NRING)],
            [pltpu.SemaphoreType.DMA for _ in range(NRING)],
            pltpu.VMEM_SHARED((NPAD, C), jnp.float32),
        ],
    )(_scatter_body)
    return deg, scat


def _deg_body(dst_hbm, out_hbm, dst_v, zbuf, ones_v, bounce, acc):
    cid = lax.axis_index("c")
    sid = lax.axis_index("s")
    wid = cid * 16 + sid
    pltpu.sync_copy(dst_hbm.at[pl.ds(wid * DEG_BPT, DEG_BPT)], dst_v)
    zero = jnp.zeros((16,), jnp.float32)
    one = jnp.full((16,), 1.0, jnp.float32)
    for j in range(8):
        zbuf[pl.ds(j * 16, 16)] = zero
    for j in range(EBLK // 16):
        ones_v[pl.ds(j * 16, 16)] = one
    base = sid * ZROWS
    pltpu.sync_copy(zbuf, acc.at[pl.ds(base, 128)])
    pltpu.sync_copy(zbuf, acc.at[pl.ds(base + 128, 128)])
    pltpu.sync_copy(zbuf, acc.at[pl.ds(base + 256, 128)])
    pltpu.sync_copy(zbuf, acc.at[pl.ds(base + 384, 128)])
    pltpu.sync_copy(zbuf.at[pl.ds(0, 120)], acc.at[pl.ds(base + 512, 120)])
    plsc.subcore_barrier()

    def body(b, _):
        pltpu.sync_copy(ones_v, acc.at[dst_v.at[b]], add=True)
        return 0

    lax.fori_loop(0, DEG_BPT, body, 0)
    plsc.subcore_barrier()
    pltpu.sync_copy(acc.at[pl.ds(base, ZROWS)], bounce)
    pltpu.sync_copy(bounce, out_hbm.at[pl.ds(cid * NPAD + base, ZROWS)])


def _scatter_body(h_hbm, src_hbm, dst_hbm, out_hbm,
                  src_v, dst_v, bufs, sems, acc):
    cid = lax.axis_index("c")
    sid = lax.axis_index("s")
    base = sid * ZROWS
    with jax.named_scope("zero_acc"):
        _fill_f32(bufs[0], EBLK, 0.0)
        _fill_f32(bufs[1], EBLK, 0.0)
        for r in range(9):
            pltpu.sync_copy(bufs[r % 2], acc.at[pl.ds(base + r * 64, 64)])
        pltpu.sync_copy(bufs[0].at[pl.ds(0, 56)],
                        acc.at[pl.ds(base + 576, 56)])
        plsc.subcore_barrier()

    nch = lax.select(cid == 0, T0 // ICH, T1 // ICH)
    blk0 = lax.select(cid == 0, sid * T0, SPLIT0 + sid * T1)

    def drain(sem, buf):
        pltpu.make_async_copy(h_hbm.at[pl.ds(0, EBLK)], buf, sem).wait()

    def chunk_body(c, _):
        pltpu.sync_copy(src_hbm.at[pl.ds(blk0 + c * ICH, ICH)], src_v)
        pltpu.sync_copy(dst_hbm.at[pl.ds(blk0 + c * ICH, ICH)], dst_v)

        def quad_body(q, _):
            b = q * NRING
            cps = []
            for k in range(NRING):
                cps.append(pltpu.async_copy(
                    h_hbm.at[src_v.at[b + k]], bufs[k], sems[k]))
            for k in range(NRING):
                cps[k].wait()
                pltpu.async_copy(bufs[k], acc.at[dst_v.at[b + k]],
                                 sems[k], add=True)
            return 0

        @pl.when(c > 0)
        def _():
            for k in range(NRING):
                drain(sems[k], bufs[k])

        def quad_drain_body(q, _):
            @pl.when(q > 0)
            def _():
                for k in range(NRING):
                    drain(sems[k], bufs[k])
            return quad_body(q, 0)

        lax.fori_loop(0, ICH // NRING, quad_drain_body, 0)
        return 0

    with jax.named_scope("edge_loop"):
        lax.fori_loop(0, nch, chunk_body, 0)
        for k in range(NRING):
            drain(sems[k], bufs[k])
        plsc.subcore_barrier()
    with jax.named_scope("writeback"):
        pltpu.sync_copy(acc.at[pl.ds(base, ZROWS)],
                        out_hbm.at[cid, pl.ds(base, ZROWS)])


RB = 2000


def _dinv_body(deg_ref, o_ref):
    d = deg_ref[0] + deg_ref[1] + 1.0
    o_ref[...] = lax.rsqrt(d)


def _dinv_call(degp):
    return pl.pallas_call(
        _dinv_body,
        out_shape=jax.ShapeDtypeStruct((NPAD // 128, 128), jnp.float32),
    )(degp.reshape(2, NPAD // 128, 128))


def _mm_body(x_ref, w_ref, dinv_ref, o_ref):
    h = jnp.dot(x_ref[...], w_ref[...], preferred_element_type=jnp.float32,
                precision=lax.Precision.HIGHEST)
    o_ref[...] = h * dinv_ref[...]


def _mm_call(x, w, dinv):
    return pl.pallas_call(
        _mm_body,
        grid=(N // RB,),
        in_specs=[
            pl.BlockSpec((RB, C), lambda i: (i, 0)),
            pl.BlockSpec((C, C), lambda i: (0, 0)),
            pl.BlockSpec((RB, 1), lambda i: (i, 0)),
        ],
        out_specs=pl.BlockSpec((RB, C), lambda i: (i, 0)),
        out_shape=jax.ShapeDtypeStruct((N, C), jnp.float32),
    )(x, w, dinv)


def _onehot(batch_blk):
    gids = lax.broadcasted_iota(jnp.int32, (1, G), 1)
    return (batch_blk == gids).astype(jnp.float32)


def _combine_body(p_ref, h_ref, dinv_ref, b_ref, batch_ref, y_ref, st_ref):
    i = pl.program_id(0)
    y = dinv_ref[...] * (p_ref[0] + p_ref[1] + h_ref[...]) + b_ref[...]
    y_ref[...] = y
    oh = _onehot(batch_ref[...])
    dn = (((0,), (0,)), ((), ()))
    sy = lax.dot_general(oh, y, dn, preferred_element_type=jnp.float32,
                         precision=lax.Precision.HIGHEST)
    sy2 = lax.dot_general(oh, y * y, dn, preferred_element_type=jnp.float32,
                          precision=lax.Precision.HIGHEST)
    cnt = lax.dot_general(oh, jnp.ones_like(y), dn,
                          preferred_element_type=jnp.float32,
                          precision=lax.Precision.HIGHEST)

    @pl.when(i == 0)
    def _():
        st_ref[...] = jnp.zeros_like(st_ref)

    st_ref[0] += sy
    st_ref[1] += sy2
    st_ref[2] += cnt


def _combine_call(part, hp, dinv, b, batch2):
    return pl.pallas_call(
        _combine_body,
        grid=(N // RB,),
        in_specs=[
            pl.BlockSpec((2, RB, C), lambda i: (0, i, 0)),
            pl.BlockSpec((RB, C), lambda i: (i, 0)),
            pl.BlockSpec((RB, 1), lambda i: (i, 0)),
            pl.BlockSpec((1, C), lambda i: (0, 0)),
            pl.BlockSpec((RB, 1), lambda i: (i, 0)),
        ],
        out_specs=[
            pl.BlockSpec((RB, C), lambda i: (i, 0)),
            pl.BlockSpec((3, G, C), lambda i: (0, 0, 0)),
        ],
        out_shape=[
            jax.ShapeDtypeStruct((N, C), jnp.float32),
            jax.ShapeDtypeStruct((3, G, C), jnp.float32),
        ],
    )(part, hp, dinv, b, batch2)


def _norm_body(y_ref, batch_ref, st_ref, gw_ref, gb_ref, gs_ref, o_ref):
    cnt = jnp.maximum(st_ref[2], 1.0)
    m = st_ref[0] / cnt
    s = gs_ref[...]
    var = st_ref[1] / cnt + m * m * (s * s - 2.0 * s)
    rstd = lax.rsqrt(var + EPS)
    a = gw_ref[...] * rstd
    c = a * m * s
    oh = _onehot(batch_ref[...])
    row_a = jnp.dot(oh, a, preferred_element_type=jnp.float32,
                    precision=lax.Precision.HIGHEST)
    row_c = jnp.dot(oh, c, preferred_element_type=jnp.float32,
                    precision=lax.Precision.HIGHEST)
    o = row_a * y_ref[...] - row_c + gb_ref[...]
    o_ref[...] = jnp.where(o >= 0, o, 0.01 * o)


def _norm_call(y, batch2, st, gw, gb, gs):
    return pl.pallas_call(
        _norm_body,
        grid=(N // RB,),
        in_specs=[
            pl.BlockSpec((RB, C), lambda i: (i, 0)),
            pl.BlockSpec((RB, 1), lambda i: (i, 0)),
            pl.BlockSpec((3, G, C), lambda i: (0, 0, 0)),
            pl.BlockSpec((1, C), lambda i: (0, 0)),
            pl.BlockSpec((1, C), lambda i: (0, 0)),
            pl.BlockSpec((1, C), lambda i: (0, 0)),
        ],
        out_specs=pl.BlockSpec((RB, C), lambda i: (i, 0)),
        out_shape=jax.ShapeDtypeStruct((N, C), jnp.float32),
    )(y, batch2, st, gw, gb, gs)


def kernel(x, edge_index, batch,
           W0, b0, gn_w0, gn_b0, gn_s0,
           W1, b1, gn_w1, gn_b1, gn_s1,
           W2, b2, gn_w2, gn_b2, gn_s2):
    src = edge_index[0].astype(jnp.int32)
    dst = edge_index[1].astype(jnp.int32)
    pad = EP - E
    srcp = jnp.concatenate([src, jnp.zeros((pad,), jnp.int32)])
    dstp = jnp.concatenate([dst, jnp.full((pad,), N, jnp.int32)])
    srcp = srcp.reshape(NBLK, EBLK)
    dstp = dstp.reshape(NBLK, EBLK)
    batch2 = batch.astype(jnp.int32).reshape(N, 1)

    deg_kernel, scatter_kernel = _sc_kernels()
    degp = deg_kernel(dstp)
    dinv = _dinv_call(degp).reshape(-1)[:N].reshape(N, 1)

    Ws = [W0, W1, W2]
    bs = [b0, b1, b2]
    gws = [gn_w0, gn_w1, gn_w2]
    gbs = [gn_b0, gn_b1, gn_b2]
    gss = [gn_s0, gn_s1, gn_s2]

    h = x
    history = []
    for i in range(3):
        hp = _mm_call(h, Ws[i], dinv)
        part = scatter_kernel(hp, srcp, dstp)
        y, st = _combine_call(part, hp, dinv, bs[i].reshape(1, C), batch2)
        h = _norm_call(y, batch2, st,
                       gws[i].reshape(1, C), gbs[i].reshape(1, C),
                       gss[i].reshape(1, C))
        history.append(h)
    return h, tuple(history)


# speedup vs baseline: 1.0939x; 1.0939x over previous
import functools

import jax
import jax.numpy as jnp
from jax import lax
from jax.experimental import pallas as pl
from jax.experimental.pallas import tpu as pltpu
from jax.experimental.pallas import tpu_sc as plsc

N = 10000
E = 320000
C = 128
G = 16
EPS = 1e-5

NPAD = 10112
EBLK = 128
NTILES = 32
EP = 327680
NBLK = EP // EBLK
DEG_BPT = NBLK // NTILES
ZROWS = NPAD // 16
ICH = 8
SPLIT0 = 1920
T0 = SPLIT0 // 16
T1 = (NBLK - SPLIT0) // 16


def _fill_f32(ref, rows, val):
    v = jnp.full((16,), val, jnp.float32)

    def body(i, _):
        for j in range(8):
            ref[i, pl.ds(j * 16, 16)] = v
        return 0

    lax.fori_loop(0, rows, body, 0)


@functools.cache
def _sc_kernels():
    mesh = plsc.VectorSubcoreMesh(core_axis_name="c", subcore_axis_name="s")
    deg = functools.partial(
        pl.kernel,
        out_type=jax.ShapeDtypeStruct((2 * NPAD,), jnp.float32),
        mesh=mesh,
        scratch_types=[
            pltpu.VMEM((DEG_BPT, EBLK), jnp.int32),
            pltpu.VMEM((128,), jnp.float32),
            pltpu.VMEM((EBLK,), jnp.float32),
            pltpu.VMEM((ZROWS,), jnp.float32),
            pltpu.VMEM_SHARED((NPAD,), jnp.float32),
        ],
    )(_deg_body)
    scat = functools.partial(
        pl.kernel,
        out_type=jax.ShapeDtypeStruct((2, NPAD, C), jnp.float32),
        mesh=mesh,
        scratch_types=[
            pltpu.VMEM((ICH, EBLK), jnp.int32),
            pltpu.VMEM((ICH, EBLK), jnp.int32),
            [pltpu.VMEM((EBLK, C), jnp.float32) for _ in range(2)],
            [pltpu.SemaphoreType.DMA for _ in range(2)],
            pltpu.VMEM_SHARED((NPAD, C), jnp.float32),
        ],
    )(_scatter_body)
    return deg, scat


def _deg_body(dst_hbm, out_hbm, dst_v, zbuf, ones_v, bounce, acc):
    cid = lax.axis_index("c")
    sid = lax.axis_index("s")
    wid = cid * 16 + sid
    pltpu.sync_copy(dst_hbm.at[pl.ds(wid * DEG_BPT, DEG_BPT)], dst_v)
    zero = jnp.zeros((16,), jnp.float32)
    one = jnp.full((16,), 1.0, jnp.float32)
    for j in range(8):
        zbuf[pl.ds(j * 16, 16)] = zero
    for j in range(EBLK // 16):
        ones_v[pl.ds(j * 16, 16)] = one
    base = sid * ZROWS
    pltpu.sync_copy(zbuf, acc.at[pl.ds(base, 128)])
    pltpu.sync_copy(zbuf, acc.at[pl.ds(base + 128, 128)])
    pltpu.sync_copy(zbuf, acc.at[pl.ds(base + 256, 128)])
    pltpu.sync_copy(zbuf, acc.at[pl.ds(base + 384, 128)])
    pltpu.sync_copy(zbuf.at[pl.ds(0, 120)], acc.at[pl.ds(base + 512, 120)])
    plsc.subcore_barrier()

    def body(b, _):
        pltpu.sync_copy(ones_v, acc.at[dst_v.at[b]], add=True)
        return 0

    lax.fori_loop(0, DEG_BPT, body, 0)
    plsc.subcore_barrier()
    pltpu.sync_copy(acc.at[pl.ds(base, ZROWS)], bounce)
    pltpu.sync_copy(bounce, out_hbm.at[pl.ds(cid * NPAD + base, ZROWS)])


def _scatter_body(h_hbm, src_hbm, dst_hbm, out_hbm,
                  src_v, dst_v, bufs, sems, acc):
    cid = lax.axis_index("c")
    sid = lax.axis_index("s")
    base = sid * ZROWS
    buf_a, buf_b = bufs
    sem_a, sem_b = sems
    with jax.named_scope("zero_acc"):
        _fill_f32(buf_a, EBLK, 0.0)
        for r in range(4):
            pltpu.sync_copy(buf_a, acc.at[pl.ds(base + r * 128, 128)])
        pltpu.sync_copy(buf_a.at[pl.ds(0, 120)],
                        acc.at[pl.ds(base + 512, 120)])
        plsc.subcore_barrier()

    nch = lax.select(cid == 0, T0 // ICH, T1 // ICH)
    blk0 = lax.select(cid == 0, sid * T0, SPLIT0 + sid * T1)

    def drain(sem, buf):
        pltpu.make_async_copy(h_hbm.at[pl.ds(0, EBLK)], buf, sem).wait()

    def chunk_body(c, _):
        @pl.when(c > 0)
        def _():
            drain(sem_a, buf_a)
            drain(sem_b, buf_b)

        pltpu.sync_copy(src_hbm.at[pl.ds(blk0 + c * ICH, ICH)], src_v)
        pltpu.sync_copy(dst_hbm.at[pl.ds(blk0 + c * ICH, ICH)], dst_v)

        def pair_body(i, _):
            @pl.when(i > 0)
            def _():
                drain(sem_a, buf_a)
                drain(sem_b, buf_b)

            b0 = 2 * i
            b1 = 2 * i + 1
            cp_a = pltpu.async_copy(h_hbm.at[src_v.at[b0]], buf_a, sem_a)
            cp_b = pltpu.async_copy(h_hbm.at[src_v.at[b1]], buf_b, sem_b)
            cp_a.wait()
            pltpu.async_copy(buf_a, acc.at[dst_v.at[b0]], sem_a, add=True)
            cp_b.wait()
            pltpu.async_copy(buf_b, acc.at[dst_v.at[b1]], sem_b, add=True)
            return 0

        lax.fori_loop(0, ICH // 2, pair_body, 0)
        return 0

    with jax.named_scope("edge_loop"):
        lax.fori_loop(0, nch, chunk_body, 0)
        drain(sem_a, buf_a)
        drain(sem_b, buf_b)
        plsc.subcore_barrier()
    with jax.named_scope("writeback"):
        pltpu.sync_copy(acc.at[pl.ds(base, ZROWS)],
                        out_hbm.at[cid, pl.ds(base, ZROWS)])


RB = 2000


def _dinv_body(deg_ref, o_ref):
    d = deg_ref[0] + deg_ref[1] + 1.0
    o_ref[...] = lax.rsqrt(d)


def _dinv_call(degp):
    return pl.pallas_call(
        _dinv_body,
        out_shape=jax.ShapeDtypeStruct((NPAD // 128, 128), jnp.float32),
    )(degp.reshape(2, NPAD // 128, 128))


def _mm_body(x_ref, w_ref, dinv_ref, o_ref):
    h = jnp.dot(x_ref[...], w_ref[...], preferred_element_type=jnp.float32,
                precision=lax.Precision.HIGHEST)
    o_ref[...] = h * dinv_ref[...]


def _mm_call(x, w, dinv):
    return pl.pallas_call(
        _mm_body,
        grid=(N // RB,),
        in_specs=[
            pl.BlockSpec((RB, C), lambda i: (i, 0)),
            pl.BlockSpec((C, C), lambda i: (0, 0)),
            pl.BlockSpec((RB, 1), lambda i: (i, 0)),
        ],
        out_specs=pl.BlockSpec((RB, C), lambda i: (i, 0)),
        out_shape=jax.ShapeDtypeStruct((N, C), jnp.float32),
    )(x, w, dinv)


def _onehot(batch_blk):
    gids = lax.broadcasted_iota(jnp.int32, (1, G), 1)
    return (batch_blk == gids).astype(jnp.float32)


def _combine_body(p_ref, h_ref, dinv_ref, b_ref, batch_ref, y_ref, st_ref):
    i = pl.program_id(0)
    y = dinv_ref[...] * (p_ref[0] + p_ref[1] + h_ref[...]) + b_ref[...]
    y_ref[...] = y
    oh = _onehot(batch_ref[...])
    dn = (((0,), (0,)), ((), ()))
    sy = lax.dot_general(oh, y, dn, preferred_element_type=jnp.float32,
                         precision=lax.Precision.HIGHEST)
    sy2 = lax.dot_general(oh, y * y, dn, preferred_element_type=jnp.float32,
                          precision=lax.Precision.HIGHEST)
    cnt = lax.dot_general(oh, jnp.ones_like(y), dn,
                          preferred_element_type=jnp.float32,
                          precision=lax.Precision.HIGHEST)

    @pl.when(i == 0)
    def _():
        st_ref[...] = jnp.zeros_like(st_ref)

    st_ref[0] += sy
    st_ref[1] += sy2
    st_ref[2] += cnt


def _combine_call(part, hp, dinv, b, batch2):
    return pl.pallas_call(
        _combine_body,
        grid=(N // RB,),
        in_specs=[
            pl.BlockSpec((2, RB, C), lambda i: (0, i, 0)),
            pl.BlockSpec((RB, C), lambda i: (i, 0)),
            pl.BlockSpec((RB, 1), lambda i: (i, 0)),
            pl.BlockSpec((1, C), lambda i: (0, 0)),
            pl.BlockSpec((RB, 1), lambda i: (i, 0)),
        ],
        out_specs=[
            pl.BlockSpec((RB, C), lambda i: (i, 0)),
            pl.BlockSpec((3, G, C), lambda i: (0, 0, 0)),
        ],
        out_shape=[
            jax.ShapeDtypeStruct((N, C), jnp.float32),
            jax.ShapeDtypeStruct((3, G, C), jnp.float32),
        ],
    )(part, hp, dinv, b, batch2)


def _norm_body(y_ref, batch_ref, st_ref, gw_ref, gb_ref, gs_ref, o_ref):
    cnt = jnp.maximum(st_ref[2], 1.0)
    m = st_ref[0] / cnt
    s = gs_ref[...]
    var = st_ref[1] / cnt + m * m * (s * s - 2.0 * s)
    rstd = lax.rsqrt(var + EPS)
    a = gw_ref[...] * rstd
    c = a * m * s
    oh = _onehot(batch_ref[...])
    row_a = jnp.dot(oh, a, preferred_element_type=jnp.float32,
                    precision=lax.Precision.HIGHEST)
    row_c = jnp.dot(oh, c, preferred_element_type=jnp.float32,
                    precision=lax.Precision.HIGHEST)
    o = row_a * y_ref[...] - row_c + gb_ref[...]
    o_ref[...] = jnp.where(o >= 0, o, 0.01 * o)


def _norm_call(y, batch2, st, gw, gb, gs):
    return pl.pallas_call(
        _norm_body,
        grid=(N // RB,),
        in_specs=[
            pl.BlockSpec((RB, C), lambda i: (i, 0)),
            pl.BlockSpec((RB, 1), lambda i: (i, 0)),
            pl.BlockSpec((3, G, C), lambda i: (0, 0, 0)),
            pl.BlockSpec((1, C), lambda i: (0, 0)),
            pl.BlockSpec((1, C), lambda i: (0, 0)),
            pl.BlockSpec((1, C), lambda i: (0, 0)),
        ],
        out_specs=pl.BlockSpec((RB, C), lambda i: (i, 0)),
        out_shape=jax.ShapeDtypeStruct((N, C), jnp.float32),
    )(y, batch2, st, gw, gb, gs)


def kernel(x, edge_index, batch,
           W0, b0, gn_w0, gn_b0, gn_s0,
           W1, b1, gn_w1, gn_b1, gn_s1,
           W2, b2, gn_w2, gn_b2, gn_s2):
    src = edge_index[0].astype(jnp.int32)
    dst = edge_index[1].astype(jnp.int32)
    pad = EP - E
    srcp = jnp.concatenate([src, jnp.zeros((pad,), jnp.int32)])
    dstp = jnp.concatenate([dst, jnp.full((pad,), N, jnp.int32)])
    srcp = srcp.reshape(NBLK, EBLK)
    dstp = dstp.reshape(NBLK, EBLK)
    batch2 = batch.astype(jnp.int32).reshape(N, 1)

    deg_kernel, scatter_kernel = _sc_kernels()
    degp = deg_kernel(dstp)
    dinv = _dinv_call(degp).reshape(-1)[:N].reshape(N, 1)

    Ws = [W0, W1, W2]
    bs = [b0, b1, b2]
    gws = [gn_w0, gn_w1, gn_w2]
    gbs = [gn_b0, gn_b1, gn_b2]
    gss = [gn_s0, gn_s1, gn_s2]

    h = x
    history = []
    for i in range(3):
        hp = _mm_call(h, Ws[i], dinv)
        part = scatter_kernel(hp, srcp, dstp)
        y, st = _combine_call(part, hp, dinv, bs[i].reshape(1, C), batch2)
        h = _norm_call(y, batch2, st,
                       gws[i].reshape(1, C), gbs[i].reshape(1, C),
                       gss[i].reshape(1, C))
        history.append(h)
    return h, tuple(history)


# speedup vs baseline: 1.1532x; 1.0543x over previous
import functools

import jax
import jax.numpy as jnp
from jax import lax
from jax.experimental import pallas as pl
from jax.experimental.pallas import tpu as pltpu
from jax.experimental.pallas import tpu_sc as plsc

N = 10000
E = 320000
C = 128
G = 16
EPS = 1e-5

NPAD = 10112
EBLK = 128
NTILES = 32
EP = 327680
NBLK = EP // EBLK
DEG_BPT = NBLK // NTILES
ZROWS = NPAD // 16
ICH = 8
SPLIT0 = 2176
T0 = SPLIT0 // 16
T1 = (NBLK - SPLIT0) // 16


def _fill_f32(ref, rows, val):
    v = jnp.full((16,), val, jnp.float32)

    def body(i, _):
        for j in range(8):
            ref[i, pl.ds(j * 16, 16)] = v
        return 0

    lax.fori_loop(0, rows, body, 0)


@functools.cache
def _sc_kernels():
    mesh = plsc.VectorSubcoreMesh(core_axis_name="c", subcore_axis_name="s")
    deg = functools.partial(
        pl.kernel,
        out_type=jax.ShapeDtypeStruct((2 * NPAD,), jnp.float32),
        mesh=mesh,
        scratch_types=[
            pltpu.VMEM((DEG_BPT, EBLK), jnp.int32),
            pltpu.VMEM((128,), jnp.float32),
            pltpu.VMEM((EBLK,), jnp.float32),
            pltpu.VMEM((ZROWS,), jnp.float32),
            pltpu.VMEM_SHARED((NPAD,), jnp.float32),
        ],
    )(_deg_body)
    scat = functools.partial(
        pl.kernel,
        out_type=jax.ShapeDtypeStruct((2, NPAD, C), jnp.float32),
        mesh=mesh,
        scratch_types=[
            pltpu.VMEM((ICH, EBLK), jnp.int32),
            pltpu.VMEM((ICH, EBLK), jnp.int32),
            [pltpu.VMEM((EBLK, C), jnp.float32) for _ in range(2)],
            [pltpu.SemaphoreType.DMA for _ in range(2)],
            pltpu.VMEM_SHARED((NPAD, C), jnp.float32),
        ],
    )(_scatter_body)
    return deg, scat


def _deg_body(dst_hbm, out_hbm, dst_v, zbuf, ones_v, bounce, acc):
    cid = lax.axis_index("c")
    sid = lax.axis_index("s")
    wid = cid * 16 + sid
    pltpu.sync_copy(dst_hbm.at[pl.ds(wid * DEG_BPT, DEG_BPT)], dst_v)
    zero = jnp.zeros((16,), jnp.float32)
    one = jnp.full((16,), 1.0, jnp.float32)
    for j in range(8):
        zbuf[pl.ds(j * 16, 16)] = zero
    for j in range(EBLK // 16):
        ones_v[pl.ds(j * 16, 16)] = one
    base = sid * ZROWS
    pltpu.sync_copy(zbuf, acc.at[pl.ds(base, 128)])
    pltpu.sync_copy(zbuf, acc.at[pl.ds(base + 128, 128)])
    pltpu.sync_copy(zbuf, acc.at[pl.ds(base + 256, 128)])
    pltpu.sync_copy(zbuf, acc.at[pl.ds(base + 384, 128)])
    pltpu.sync_copy(zbuf.at[pl.ds(0, 120)], acc.at[pl.ds(base + 512, 120)])
    plsc.subcore_barrier()

    def body(b, _):
        pltpu.sync_copy(ones_v, acc.at[dst_v.at[b]], add=True)
        return 0

    lax.fori_loop(0, DEG_BPT, body, 0)
    plsc.subcore_barrier()
    pltpu.sync_copy(acc.at[pl.ds(base, ZROWS)], bounce)
    pltpu.sync_copy(bounce, out_hbm.at[pl.ds(cid * NPAD + base, ZROWS)])


def _scatter_body(h_hbm, src_hbm, dst_hbm, out_hbm,
                  src_v, dst_v, bufs, sems, acc):
    cid = lax.axis_index("c")
    sid = lax.axis_index("s")
    base = sid * ZROWS
    buf_a, buf_b = bufs
    sem_a, sem_b = sems
    with jax.named_scope("zero_acc"):
        _fill_f32(buf_a, EBLK, 0.0)
        for r in range(4):
            pltpu.sync_copy(buf_a, acc.at[pl.ds(base + r * 128, 128)])
        pltpu.sync_copy(buf_a.at[pl.ds(0, 120)],
                        acc.at[pl.ds(base + 512, 120)])
        plsc.subcore_barrier()

    nch = lax.select(cid == 0, T0 // ICH, T1 // ICH)
    blk0 = lax.select(cid == 0, sid * T0, SPLIT0 + sid * T1)

    def drain(sem, buf):
        pltpu.make_async_copy(h_hbm.at[pl.ds(0, EBLK)], buf, sem).wait()

    def chunk_body(c, _):
        @pl.when(c > 0)
        def _():
            drain(sem_a, buf_a)
            drain(sem_b, buf_b)

        pltpu.sync_copy(src_hbm.at[pl.ds(blk0 + c * ICH, ICH)], src_v)
        pltpu.sync_copy(dst_hbm.at[pl.ds(blk0 + c * ICH, ICH)], dst_v)

        def pair_body(i, _):
            @pl.when(i > 0)
            def _():
                drain(sem_a, buf_a)
                drain(sem_b, buf_b)

            b0 = 2 * i
            b1 = 2 * i + 1
            cp_a = pltpu.async_copy(h_hbm.at[src_v.at[b0]], buf_a, sem_a)
            cp_b = pltpu.async_copy(h_hbm.at[src_v.at[b1]], buf_b, sem_b)
            cp_a.wait()
            pltpu.async_copy(buf_a, acc.at[dst_v.at[b0]], sem_a, add=True)
            cp_b.wait()
            pltpu.async_copy(buf_b, acc.at[dst_v.at[b1]], sem_b, add=True)
            return 0

        lax.fori_loop(0, ICH // 2, pair_body, 0)
        return 0

    with jax.named_scope("edge_loop"):
        lax.fori_loop(0, nch, chunk_body, 0)
        drain(sem_a, buf_a)
        drain(sem_b, buf_b)
        plsc.subcore_barrier()
    with jax.named_scope("writeback"):
        pltpu.sync_copy(acc.at[pl.ds(base, ZROWS)],
                        out_hbm.at[cid, pl.ds(base, ZROWS)])


RB = 2000


def _dinv_body(deg_ref, o_ref):
    d = deg_ref[0] + deg_ref[1] + 1.0
    o_ref[...] = lax.rsqrt(d)


def _dinv_call(degp):
    return pl.pallas_call(
        _dinv_body,
        out_shape=jax.ShapeDtypeStruct((NPAD // 128, 128), jnp.float32),
    )(degp.reshape(2, NPAD // 128, 128))


def _mm_body(x_ref, w_ref, dinv_ref, o_ref):
    h = jnp.dot(x_ref[...], w_ref[...], preferred_element_type=jnp.float32,
                precision=lax.Precision.HIGHEST)
    o_ref[...] = h * dinv_ref[...]


def _mm_call(x, w, dinv):
    return pl.pallas_call(
        _mm_body,
        grid=(N // RB,),
        in_specs=[
            pl.BlockSpec((RB, C), lambda i: (i, 0)),
            pl.BlockSpec((C, C), lambda i: (0, 0)),
            pl.BlockSpec((RB, 1), lambda i: (i, 0)),
        ],
        out_specs=pl.BlockSpec((RB, C), lambda i: (i, 0)),
        out_shape=jax.ShapeDtypeStruct((N, C), jnp.float32),
    )(x, w, dinv)


def _onehot(batch_blk):
    gids = lax.broadcasted_iota(jnp.int32, (1, G), 1)
    return (batch_blk == gids).astype(jnp.float32)


def _combine_body(p_ref, h_ref, dinv_ref, b_ref, batch_ref, y_ref, st_ref):
    i = pl.program_id(0)
    y = dinv_ref[...] * (p_ref[0] + p_ref[1] + h_ref[...]) + b_ref[...]
    y_ref[...] = y
    oh = _onehot(batch_ref[...])
    dn = (((0,), (0,)), ((), ()))
    sy = lax.dot_general(oh, y, dn, preferred_element_type=jnp.float32,
                         precision=lax.Precision.HIGHEST)
    sy2 = lax.dot_general(oh, y * y, dn, preferred_element_type=jnp.float32,
                          precision=lax.Precision.HIGHEST)
    cnt = lax.dot_general(oh, jnp.ones_like(y), dn,
                          preferred_element_type=jnp.float32,
                          precision=lax.Precision.HIGHEST)

    @pl.when(i == 0)
    def _():
        st_ref[...] = jnp.zeros_like(st_ref)

    st_ref[0] += sy
    st_ref[1] += sy2
    st_ref[2] += cnt


def _combine_call(part, hp, dinv, b, batch2):
    return pl.pallas_call(
        _combine_body,
        grid=(N // RB,),
        in_specs=[
            pl.BlockSpec((2, RB, C), lambda i: (0, i, 0)),
            pl.BlockSpec((RB, C), lambda i: (i, 0)),
            pl.BlockSpec((RB, 1), lambda i: (i, 0)),
            pl.BlockSpec((1, C), lambda i: (0, 0)),
            pl.BlockSpec((RB, 1), lambda i: (i, 0)),
        ],
        out_specs=[
            pl.BlockSpec((RB, C), lambda i: (i, 0)),
            pl.BlockSpec((3, G, C), lambda i: (0, 0, 0)),
        ],
        out_shape=[
            jax.ShapeDtypeStruct((N, C), jnp.float32),
            jax.ShapeDtypeStruct((3, G, C), jnp.float32),
        ],
    )(part, hp, dinv, b, batch2)


def _norm_body(y_ref, batch_ref, st_ref, gw_ref, gb_ref, gs_ref, o_ref):
    cnt = jnp.maximum(st_ref[2], 1.0)
    m = st_ref[0] / cnt
    s = gs_ref[...]
    var = st_ref[1] / cnt + m * m * (s * s - 2.0 * s)
    rstd = lax.rsqrt(var + EPS)
    a = gw_ref[...] * rstd
    c = a * m * s
    oh = _onehot(batch_ref[...])
    row_a = jnp.dot(oh, a, preferred_element_type=jnp.float32,
                    precision=lax.Precision.HIGHEST)
    row_c = jnp.dot(oh, c, preferred_element_type=jnp.float32,
                    precision=lax.Precision.HIGHEST)
    o = row_a * y_ref[...] - row_c + gb_ref[...]
    o_ref[...] = jnp.where(o >= 0, o, 0.01 * o)


def _norm_call(y, batch2, st, gw, gb, gs):
    return pl.pallas_call(
        _norm_body,
        grid=(N // RB,),
        in_specs=[
            pl.BlockSpec((RB, C), lambda i: (i, 0)),
            pl.BlockSpec((RB, 1), lambda i: (i, 0)),
            pl.BlockSpec((3, G, C), lambda i: (0, 0, 0)),
            pl.BlockSpec((1, C), lambda i: (0, 0)),
            pl.BlockSpec((1, C), lambda i: (0, 0)),
            pl.BlockSpec((1, C), lambda i: (0, 0)),
        ],
        out_specs=pl.BlockSpec((RB, C), lambda i: (i, 0)),
        out_shape=jax.ShapeDtypeStruct((N, C), jnp.float32),
    )(y, batch2, st, gw, gb, gs)


def kernel(x, edge_index, batch,
           W0, b0, gn_w0, gn_b0, gn_s0,
           W1, b1, gn_w1, gn_b1, gn_s1,
           W2, b2, gn_w2, gn_b2, gn_s2):
    src = edge_index[0].astype(jnp.int32)
    dst = edge_index[1].astype(jnp.int32)
    pad = EP - E
    srcp = jnp.concatenate([src, jnp.zeros((pad,), jnp.int32)])
    dstp = jnp.concatenate([dst, jnp.full((pad,), N, jnp.int32)])
    srcp = srcp.reshape(NBLK, EBLK)
    dstp = dstp.reshape(NBLK, EBLK)
    batch2 = batch.astype(jnp.int32).reshape(N, 1)

    deg_kernel, scatter_kernel = _sc_kernels()
    degp = deg_kernel(dstp)
    dinv = _dinv_call(degp).reshape(-1)[:N].reshape(N, 1)

    Ws = [W0, W1, W2]
    bs = [b0, b1, b2]
    gws = [gn_w0, gn_w1, gn_w2]
    gbs = [gn_b0, gn_b1, gn_b2]
    gss = [gn_s0, gn_s1, gn_s2]

    h = x
    history = []
    for i in range(3):
        hp = _mm_call(h, Ws[i], dinv)
        part = scatter_kernel(hp, srcp, dstp)
        y, st = _combine_call(part, hp, dinv, bs[i].reshape(1, C), batch2)
        h = _norm_call(y, batch2, st,
                       gws[i].reshape(1, C), gbs[i].reshape(1, C),
                       gss[i].reshape(1, C))
        history.append(h)
    return h, tuple(history)


# speedup vs baseline: 1.2112x; 1.0503x over previous
import functools

import jax
import jax.numpy as jnp
from jax import lax
from jax.experimental import pallas as pl
from jax.experimental.pallas import tpu as pltpu
from jax.experimental.pallas import tpu_sc as plsc

N = 10000
E = 320000
C = 128
G = 16
EPS = 1e-5

NPAD = 10112
EBLK = 128
NTILES = 32
EP = 327680
NBLK = EP // EBLK
DEG_BPT = NBLK // NTILES
ZROWS = NPAD // 16
ICH = 8
SPLIT0 = 2432
T0 = SPLIT0 // 16
T1 = (NBLK - SPLIT0) // 16


def _fill_f32(ref, rows, val):
    v = jnp.full((16,), val, jnp.float32)

    def body(i, _):
        for j in range(8):
            ref[i, pl.ds(j * 16, 16)] = v
        return 0

    lax.fori_loop(0, rows, body, 0)


@functools.cache
def _sc_kernels():
    mesh = plsc.VectorSubcoreMesh(core_axis_name="c", subcore_axis_name="s")
    deg = functools.partial(
        pl.kernel,
        out_type=jax.ShapeDtypeStruct((2 * NPAD,), jnp.float32),
        mesh=mesh,
        scratch_types=[
            pltpu.VMEM((DEG_BPT, EBLK), jnp.int32),
            pltpu.VMEM((128,), jnp.float32),
            pltpu.VMEM((EBLK,), jnp.float32),
            pltpu.VMEM((ZROWS,), jnp.float32),
            pltpu.VMEM_SHARED((NPAD,), jnp.float32),
        ],
    )(_deg_body)
    scat = functools.partial(
        pl.kernel,
        out_type=jax.ShapeDtypeStruct((2, NPAD, C), jnp.float32),
        mesh=mesh,
        scratch_types=[
            pltpu.VMEM((ICH, EBLK), jnp.int32),
            pltpu.VMEM((ICH, EBLK), jnp.int32),
            [pltpu.VMEM((EBLK, C), jnp.float32) for _ in range(2)],
            [pltpu.SemaphoreType.DMA for _ in range(2)],
            pltpu.VMEM_SHARED((NPAD, C), jnp.float32),
        ],
    )(_scatter_body)
    return deg, scat


def _deg_body(dst_hbm, out_hbm, dst_v, zbuf, ones_v, bounce, acc):
    cid = lax.axis_index("c")
    sid = lax.axis_index("s")
    wid = cid * 16 + sid
    pltpu.sync_copy(dst_hbm.at[pl.ds(wid * DEG_BPT, DEG_BPT)], dst_v)
    zero = jnp.zeros((16,), jnp.float32)
    one = jnp.full((16,), 1.0, jnp.float32)
    for j in range(8):
        zbuf[pl.ds(j * 16, 16)] = zero
    for j in range(EBLK // 16):
        ones_v[pl.ds(j * 16, 16)] = one
    base = sid * ZROWS
    pltpu.sync_copy(zbuf, acc.at[pl.ds(base, 128)])
    pltpu.sync_copy(zbuf, acc.at[pl.ds(base + 128, 128)])
    pltpu.sync_copy(zbuf, acc.at[pl.ds(base + 256, 128)])
    pltpu.sync_copy(zbuf, acc.at[pl.ds(base + 384, 128)])
    pltpu.sync_copy(zbuf.at[pl.ds(0, 120)], acc.at[pl.ds(base + 512, 120)])
    plsc.subcore_barrier()

    def body(b, _):
        pltpu.sync_copy(ones_v, acc.at[dst_v.at[b]], add=True)
        return 0

    lax.fori_loop(0, DEG_BPT, body, 0)
    plsc.subcore_barrier()
    pltpu.sync_copy(acc.at[pl.ds(base, ZROWS)], bounce)
    pltpu.sync_copy(bounce, out_hbm.at[pl.ds(cid * NPAD + base, ZROWS)])


def _scatter_body(h_hbm, src_hbm, dst_hbm, out_hbm,
                  src_v, dst_v, bufs, sems, acc):
    cid = lax.axis_index("c")
    sid = lax.axis_index("s")
    base = sid * ZROWS
    buf_a, buf_b = bufs
    sem_a, sem_b = sems
    with jax.named_scope("zero_acc"):
        _fill_f32(buf_a, EBLK, 0.0)
        for r in range(4):
            pltpu.sync_copy(buf_a, acc.at[pl.ds(base + r * 128, 128)])
        pltpu.sync_copy(buf_a.at[pl.ds(0, 120)],
                        acc.at[pl.ds(base + 512, 120)])
        plsc.subcore_barrier()

    nch = lax.select(cid == 0, T0 // ICH, T1 // ICH)
    blk0 = lax.select(cid == 0, sid * T0, SPLIT0 + sid * T1)

    def drain(sem, buf):
        pltpu.make_async_copy(h_hbm.at[pl.ds(0, EBLK)], buf, sem).wait()

    def chunk_body(c, _):
        @pl.when(c > 0)
        def _():
            drain(sem_a, buf_a)
            drain(sem_b, buf_b)

        pltpu.sync_copy(src_hbm.at[pl.ds(blk0 + c * ICH, ICH)], src_v)
        pltpu.sync_copy(dst_hbm.at[pl.ds(blk0 + c * ICH, ICH)], dst_v)

        def pair_body(i, _):
            @pl.when(i > 0)
            def _():
                drain(sem_a, buf_a)
                drain(sem_b, buf_b)

            b0 = 2 * i
            b1 = 2 * i + 1
            cp_a = pltpu.async_copy(h_hbm.at[src_v.at[b0]], buf_a, sem_a)
            cp_b = pltpu.async_copy(h_hbm.at[src_v.at[b1]], buf_b, sem_b)
            cp_a.wait()
            pltpu.async_copy(buf_a, acc.at[dst_v.at[b0]], sem_a, add=True)
            cp_b.wait()
            pltpu.async_copy(buf_b, acc.at[dst_v.at[b1]], sem_b, add=True)
            return 0

        lax.fori_loop(0, ICH // 2, pair_body, 0)
        return 0

    with jax.named_scope("edge_loop"):
        lax.fori_loop(0, nch, chunk_body, 0)
        drain(sem_a, buf_a)
        drain(sem_b, buf_b)
        plsc.subcore_barrier()
    with jax.named_scope("writeback"):
        pltpu.sync_copy(acc.at[pl.ds(base, ZROWS)],
                        out_hbm.at[cid, pl.ds(base, ZROWS)])


RB = 2000


def _dinv_body(deg_ref, o_ref):
    d = deg_ref[0] + deg_ref[1] + 1.0
    o_ref[...] = lax.rsqrt(d)


def _dinv_call(degp):
    return pl.pallas_call(
        _dinv_body,
        out_shape=jax.ShapeDtypeStruct((NPAD // 128, 128), jnp.float32),
    )(degp.reshape(2, NPAD // 128, 128))


def _mm_body(x_ref, w_ref, dinv_ref, o_ref):
    h = jnp.dot(x_ref[...], w_ref[...], preferred_element_type=jnp.float32,
                precision=lax.Precision.HIGHEST)
    o_ref[...] = h * dinv_ref[...]


def _mm_call(x, w, dinv):
    return pl.pallas_call(
        _mm_body,
        grid=(N // RB,),
        in_specs=[
            pl.BlockSpec((RB, C), lambda i: (i, 0)),
            pl.BlockSpec((C, C), lambda i: (0, 0)),
            pl.BlockSpec((RB, 1), lambda i: (i, 0)),
        ],
        out_specs=pl.BlockSpec((RB, C), lambda i: (i, 0)),
        out_shape=jax.ShapeDtypeStruct((N, C), jnp.float32),
    )(x, w, dinv)


def _onehot(batch_blk):
    gids = lax.broadcasted_iota(jnp.int32, (1, G), 1)
    return (batch_blk == gids).astype(jnp.float32)


def _combine_body(p_ref, h_ref, dinv_ref, b_ref, batch_ref, y_ref, st_ref):
    i = pl.program_id(0)
    y = dinv_ref[...] * (p_ref[0] + p_ref[1] + h_ref[...]) + b_ref[...]
    y_ref[...] = y
    oh = _onehot(batch_ref[...])
    dn = (((0,), (0,)), ((), ()))
    sy = lax.dot_general(oh, y, dn, preferred_element_type=jnp.float32,
                         precision=lax.Precision.HIGHEST)
    sy2 = lax.dot_general(oh, y * y, dn, preferred_element_type=jnp.float32,
                          precision=lax.Precision.HIGHEST)
    cnt = lax.dot_general(oh, jnp.ones_like(y), dn,
                          preferred_element_type=jnp.float32,
                          precision=lax.Precision.HIGHEST)

    @pl.when(i == 0)
    def _():
        st_ref[...] = jnp.zeros_like(st_ref)

    st_ref[0] += sy
    st_ref[1] += sy2
    st_ref[2] += cnt


def _combine_call(part, hp, dinv, b, batch2):
    return pl.pallas_call(
        _combine_body,
        grid=(N // RB,),
        in_specs=[
            pl.BlockSpec((2, RB, C), lambda i: (0, i, 0)),
            pl.BlockSpec((RB, C), lambda i: (i, 0)),
            pl.BlockSpec((RB, 1), lambda i: (i, 0)),
            pl.BlockSpec((1, C), lambda i: (0, 0)),
            pl.BlockSpec((RB, 1), lambda i: (i, 0)),
        ],
        out_specs=[
            pl.BlockSpec((RB, C), lambda i: (i, 0)),
            pl.BlockSpec((3, G, C), lambda i: (0, 0, 0)),
        ],
        out_shape=[
            jax.ShapeDtypeStruct((N, C), jnp.float32),
            jax.ShapeDtypeStruct((3, G, C), jnp.float32),
        ],
    )(part, hp, dinv, b, batch2)


def _norm_body(y_ref, batch_ref, st_ref, gw_ref, gb_ref, gs_ref, o_ref):
    cnt = jnp.maximum(st_ref[2], 1.0)
    m = st_ref[0] / cnt
    s = gs_ref[...]
    var = st_ref[1] / cnt + m * m * (s * s - 2.0 * s)
    rstd = lax.rsqrt(var + EPS)
    a = gw_ref[...] * rstd
    c = a * m * s
    oh = _onehot(batch_ref[...])
    row_a = jnp.dot(oh, a, preferred_element_type=jnp.float32,
                    precision=lax.Precision.HIGHEST)
    row_c = jnp.dot(oh, c, preferred_element_type=jnp.float32,
                    precision=lax.Precision.HIGHEST)
    o = row_a * y_ref[...] - row_c + gb_ref[...]
    o_ref[...] = jnp.where(o >= 0, o, 0.01 * o)


def _norm_call(y, batch2, st, gw, gb, gs):
    return pl.pallas_call(
        _norm_body,
        grid=(N // RB,),
        in_specs=[
            pl.BlockSpec((RB, C), lambda i: (i, 0)),
            pl.BlockSpec((RB, 1), lambda i: (i, 0)),
            pl.BlockSpec((3, G, C), lambda i: (0, 0, 0)),
            pl.BlockSpec((1, C), lambda i: (0, 0)),
            pl.BlockSpec((1, C), lambda i: (0, 0)),
            pl.BlockSpec((1, C), lambda i: (0, 0)),
        ],
        out_specs=pl.BlockSpec((RB, C), lambda i: (i, 0)),
        out_shape=jax.ShapeDtypeStruct((N, C), jnp.float32),
    )(y, batch2, st, gw, gb, gs)


def kernel(x, edge_index, batch,
           W0, b0, gn_w0, gn_b0, gn_s0,
           W1, b1, gn_w1, gn_b1, gn_s1,
           W2, b2, gn_w2, gn_b2, gn_s2):
    src = edge_index[0].astype(jnp.int32)
    dst = edge_index[1].astype(jnp.int32)
    pad = EP - E
    srcp = jnp.concatenate([src, jnp.zeros((pad,), jnp.int32)])
    dstp = jnp.concatenate([dst, jnp.full((pad,), N, jnp.int32)])
    srcp = srcp.reshape(NBLK, EBLK)
    dstp = dstp.reshape(NBLK, EBLK)
    batch2 = batch.astype(jnp.int32).reshape(N, 1)

    deg_kernel, scatter_kernel = _sc_kernels()
    degp = deg_kernel(dstp)
    dinv = _dinv_call(degp).reshape(-1)[:N].reshape(N, 1)

    Ws = [W0, W1, W2]
    bs = [b0, b1, b2]
    gws = [gn_w0, gn_w1, gn_w2]
    gbs = [gn_b0, gn_b1, gn_b2]
    gss = [gn_s0, gn_s1, gn_s2]

    h = x
    history = []
    for i in range(3):
        hp = _mm_call(h, Ws[i], dinv)
        part = scatter_kernel(hp, srcp, dstp)
        y, st = _combine_call(part, hp, dinv, bs[i].reshape(1, C), batch2)
        h = _norm_call(y, batch2, st,
                       gws[i].reshape(1, C), gbs[i].reshape(1, C),
                       gss[i].reshape(1, C))
        history.append(h)
    return h, tuple(history)
